# baseline (device time: 28742 ns/iter reference)
import jax
import jax.numpy as jnp
from jax import lax
from jax.experimental import pallas as pl
from jax.experimental.pallas import tpu as pltpu

N_DEV = 4
EXPERTS_PER_DEV = 2


def kernel(x, router_W, route_idx, expert_W):
    del router_W
    n_tok, _ = x.shape
    _, _, h = expert_W.shape

    def body(x_ref, idx_ref, w_ref, out_ref, comm_ref, send_sems, recv_sems):
        my_pos = lax.axis_index("i")
        left = lax.rem(my_pos + N_DEV - 1, N_DEV)
        right = lax.rem(my_pos + 1, N_DEV)

        barrier_sem = pltpu.get_barrier_semaphore()
        for nbr in (left, right):
            pl.semaphore_signal(
                barrier_sem, inc=1,
                device_id=(nbr,), device_id_type=pl.DeviceIdType.MESH,
            )
        pl.semaphore_wait(barrier_sem, 2)

        idx = idx_ref[...]
        e0 = my_pos * EXPERTS_PER_DEV
        acc = jnp.zeros((n_tok, h), jnp.float32)
        for k in range(EXPERTS_PER_DEV):
            mask = idx == (e0 + k)
            xk = jnp.where(mask, x_ref[...], 0.0).astype(jnp.bfloat16)
            wk = w_ref[k, :, :].astype(jnp.bfloat16)
            acc = acc + lax.dot(xk, wk, preferred_element_type=jnp.float32)

        out_ref[...] = acc
        comm_ref[0, :, :] = acc.astype(jnp.bfloat16)

        for hop in range(N_DEV - 1):
            rdma = pltpu.make_async_remote_copy(
                src_ref=comm_ref.at[hop],
                dst_ref=comm_ref.at[hop + 1],
                send_sem=send_sems.at[hop],
                recv_sem=recv_sems.at[hop],
                device_id=(right,),
                device_id_type=pl.DeviceIdType.MESH,
            )
            rdma.start()
            rdma.wait()
            out_ref[...] += comm_ref[hop + 1, :, :].astype(jnp.float32)

    return pl.pallas_call(
        body,
        out_shape=jax.ShapeDtypeStruct((n_tok, h), jnp.float32),
        in_specs=[pl.BlockSpec(memory_space=pltpu.VMEM)] * 3,
        out_specs=pl.BlockSpec(memory_space=pltpu.VMEM),
        scratch_shapes=[
            pltpu.VMEM((N_DEV, n_tok, h), jnp.bfloat16),
            pltpu.SemaphoreType.DMA((N_DEV - 1,)),
            pltpu.SemaphoreType.DMA((N_DEV - 1,)),
        ],
        compiler_params=pltpu.CompilerParams(collective_id=0),
    )(x, route_idx, expert_W)


# device time: 17742 ns/iter; 1.6200x vs baseline; 1.6200x over previous
import jax
import jax.numpy as jnp
from jax import lax
from jax.experimental import pallas as pl
from jax.experimental.pallas import tpu as pltpu

N_DEV = 4
EXPERTS_PER_DEV = 2
NC = 4


def kernel(x, router_W, route_idx, expert_W):
    del router_W
    n_tok, _ = x.shape
    _, _, h = expert_W.shape
    rpc = n_tok // NC

    def body(x_ref, idx_ref, w_ref, out_ref,
             send1, recv1, send2, recv2, ss1, rs1, ss2, rs2):
        my_pos = lax.axis_index("i")
        pA = jnp.bitwise_xor(my_pos, 1)
        pB = 3 - my_pos

        barrier_sem = pltpu.get_barrier_semaphore()
        for nbr in (pA, pB):
            pl.semaphore_signal(
                barrier_sem, inc=1,
                device_id=(nbr,), device_id_type=pl.DeviceIdType.MESH,
            )
        pl.semaphore_wait(barrier_sem, 2)

        idx = idx_ref[...]
        e0 = my_pos * EXPERTS_PER_DEV
        xf = x_ref[...]
        x0 = jnp.where(idx == e0, xf, 0.0).astype(jnp.bfloat16)
        x1 = jnp.where(idx == e0 + 1, xf, 0.0).astype(jnp.bfloat16)
        w0 = w_ref[0, :, :].astype(jnp.bfloat16)
        w1 = w_ref[1, :, :].astype(jnp.bfloat16)

        rdma1 = []
        for c in range(NC):
            rs = slice(c * rpc, (c + 1) * rpc)
            acc_c = (
                lax.dot(x0[rs, :], w0, preferred_element_type=jnp.float32)
                + lax.dot(x1[rs, :], w1, preferred_element_type=jnp.float32)
            )
            send1[c, :, :] = acc_c.astype(jnp.bfloat16)
            d = pltpu.make_async_remote_copy(
                src_ref=send1.at[c], dst_ref=recv1.at[c],
                send_sem=ss1.at[c], recv_sem=rs1.at[c],
                device_id=(pA,), device_id_type=pl.DeviceIdType.MESH,
            )
            d.start()
            rdma1.append(d)

        rdma2 = []
        for c in range(NC):
            rdma1[c].wait_recv()
            send2[c, :, :] = send1[c, :, :] + recv1[c, :, :]
            d = pltpu.make_async_remote_copy(
                src_ref=send2.at[c], dst_ref=recv2.at[c],
                send_sem=ss2.at[c], recv_sem=rs2.at[c],
                device_id=(pB,), device_id_type=pl.DeviceIdType.MESH,
            )
            d.start()
            rdma2.append(d)

        for c in range(NC):
            rdma2[c].wait_recv()
            out_ref[pl.ds(c * rpc, rpc), :] = (
                send2[c, :, :] + recv2[c, :, :]
            ).astype(jnp.float32)

        for c in range(NC):
            rdma1[c].wait_send()
            rdma2[c].wait_send()

    return pl.pallas_call(
        body,
        out_shape=jax.ShapeDtypeStruct((n_tok, h), jnp.float32),
        in_specs=[pl.BlockSpec(memory_space=pltpu.VMEM)] * 3,
        out_specs=pl.BlockSpec(memory_space=pltpu.VMEM),
        scratch_shapes=[
            pltpu.VMEM((NC, rpc, h), jnp.bfloat16),
            pltpu.VMEM((NC, rpc, h), jnp.bfloat16),
            pltpu.VMEM((NC, rpc, h), jnp.bfloat16),
            pltpu.VMEM((NC, rpc, h), jnp.bfloat16),
            pltpu.SemaphoreType.DMA((NC,)),
            pltpu.SemaphoreType.DMA((NC,)),
            pltpu.SemaphoreType.DMA((NC,)),
            pltpu.SemaphoreType.DMA((NC,)),
        ],
        compiler_params=pltpu.CompilerParams(collective_id=0),
    )(x, route_idx, expert_W)
